# baseline (speedup 1.0000x reference)
import jax
import jax.numpy as jnp
import numpy as np
from jax.experimental import pallas as pl

N = 10000
E = 320000
F = 128
RATIO = 0.3
NEG_SLOPE = 0.2
EPS = 1e-8


def _layer_norm(v, g, b, eps=1e-5):
    mu = jnp.mean(v, axis=-1, keepdims=True)
    var = jnp.var(v, axis=-1, keepdims=True)
    return (v - mu) / jnp.sqrt(var + eps) * g + b


def _silu(v):
    return v * jax.nn.sigmoid(v)


def _scale_body(xg_ref, tv_ref, o_ref):
    o_ref[...] = xg_ref[...] * tv_ref[...]


def kernel(x, pos, edge_index, Wq, bq, Watt, batt, le_w, le_W1, le_b1, le_W2, le_b2, m_W1, m_b1, m_W2, m_b2, a_W, a_b, u_W1, u_b1, u_W2, u_b2, p_W1, p_b1, p_W2, ln_g, ln_b, coors_scale):
    src = edge_index[0]; dst = edge_index[1]
    x_i = x[dst]; x_j = x[src]
    pos_dir = pos[dst] - pos[src]
    dist = jnp.linalg.norm(pos_dir, axis=-1, keepdims=True)
    m_in = jnp.concatenate([_layer_norm(x_i, ln_g, ln_b), _layer_norm(x_j, ln_g, ln_b), dist], axis=-1)
    h = _silu(m_in @ m_W1 + m_b1)
    msg = _silu(h @ m_W2 + m_b2)
    msg = jax.nn.sigmoid(msg @ a_W + a_b) * msg
    agg_node = jax.ops.segment_sum(msg, dst, num_segments=N)
    x_pool = _silu(jnp.concatenate([x, agg_node], axis=-1) @ u_W1 + u_b1) @ u_W2 + u_b2 + x
    i = src; j = dst
    x_pool_j = x_pool[j]
    X_q = jax.ops.segment_max(x_pool_j, i, num_segments=N)
    X_q = jnp.where(jnp.isfinite(X_q), X_q, 0.0)
    M_q = (X_q @ Wq + bq)[i]
    score = (jnp.concatenate([M_q, x_pool_j], axis=-1) @ Watt + batt)[:, 0]
    score = jax.nn.leaky_relu(score, NEG_SLOPE)
    smax = jax.ops.segment_max(score, i, num_segments=N)
    smax = jnp.where(jnp.isfinite(smax), smax, 0.0)
    ex = jnp.exp(score - smax[i])
    denom = jax.ops.segment_sum(ex, i, num_segments=N)
    attn = ex / (denom[i] + 1e-16)
    x_agg = jax.ops.segment_sum(x[j] * attn[:, None], i, num_segments=N)
    ew = jnp.where(i == j, 0.0, 1.0).astype(x.dtype)
    deg = jax.ops.segment_sum(ew, i, num_segments=N)
    aggr = jax.ops.segment_sum(ew[:, None] * (x_agg @ le_w)[j], i, num_segments=N)
    le_out = deg[:, None] * (x_agg @ le_W1 + le_b1) + aggr + (x_agg @ le_W2 + le_b2)
    fitness = jax.nn.sigmoid(le_out)[:, 0]
    k = int(np.ceil(RATIO * N))
    topv, perm = jax.lax.top_k(fitness, k)
    x_out = pl.pallas_call(
        _scale_body,
        out_shape=jax.ShapeDtypeStruct((k, F), jnp.float32),
    )(x_agg[perm], topv[:, None])
    return x_out, fitness, perm


# speedup vs baseline: 1.3175x; 1.3175x over previous
import functools

import jax
import jax.numpy as jnp
import numpy as np
from jax import lax
from jax.experimental import pallas as pl
from jax.experimental.pallas import tpu as pltpu
from jax.experimental.pallas import tpu_sc as plsc

N = 10000
E = 320000
F = 128
RATIO = 0.3
NEG_SLOPE = 0.2
EPS = 1e-8

_NC = 2
_NS = 16
_NW = _NC * _NS
_C = 128


def _layer_norm(v, g, b, eps=1e-5):
    mu = jnp.mean(v, axis=-1, keepdims=True)
    var = jnp.var(v, axis=-1, keepdims=True)
    return (v - mu) / jnp.sqrt(var + eps) * g + b


def _silu(v):
    return v * jax.nn.sigmoid(v)


@functools.partial(jax.jit, static_argnums=(2,))
def _sc_gather_rows(table, idx, n_out):
    rows, D = table.shape
    nblocks = n_out // _C
    iters = (nblocks + _NW - 1) // _NW
    mesh = plsc.VectorSubcoreMesh(core_axis_name="c", subcore_axis_name="s")

    def body(table_hbm, idx_hbm, out_hbm, idx_v, rows_v, sem):
        wid = lax.axis_index("s") * _NC + lax.axis_index("c")

        def step(g, carry):
            blk = g * _NW + wid

            @pl.when(blk < nblocks)
            def _():
                off = blk * _C
                pltpu.sync_copy(idx_hbm.at[pl.ds(off, _C)], idx_v)
                pltpu.async_copy(table_hbm.at[idx_v], rows_v, sem).wait()
                pltpu.sync_copy(rows_v, out_hbm.at[pl.ds(off, _C)])

            return carry

        lax.fori_loop(0, iters, step, 0)

    return pl.kernel(
        body,
        out_type=jax.ShapeDtypeStruct((n_out, D), jnp.float32),
        mesh=mesh,
        scratch_types=[
            pltpu.VMEM((_C,), jnp.int32),
            pltpu.VMEM((_C, D), jnp.float32),
            pltpu.SemaphoreType.DMA,
        ],
    )(table, idx)


def _scale_body(xg_ref, tv_ref, o_ref):
    o_ref[...] = xg_ref[...] * tv_ref[...]


def kernel(x, pos, edge_index, Wq, bq, Watt, batt, le_w, le_W1, le_b1, le_W2, le_b2, m_W1, m_b1, m_W2, m_b2, a_W, a_b, u_W1, u_b1, u_W2, u_b2, p_W1, p_b1, p_W2, ln_g, ln_b, coors_scale):
    src = edge_index[0]; dst = edge_index[1]

    ln_x = _layer_norm(x, ln_g, ln_b)
    t1 = jnp.concatenate([ln_x, pos, jnp.zeros((N, 125), jnp.float32)], axis=1)
    g1d = _sc_gather_rows(t1, dst, E)
    g1s = _sc_gather_rows(t1, src, E)
    ln_i = g1d[:, :F]; ln_j = g1s[:, :F]
    pos_dir = g1d[:, F:F + 3] - g1s[:, F:F + 3]
    dist = jnp.linalg.norm(pos_dir, axis=-1, keepdims=True)
    m_in = jnp.concatenate([ln_i, ln_j, dist], axis=-1)
    h = _silu(m_in @ m_W1 + m_b1)
    msg = _silu(h @ m_W2 + m_b2)
    msg = jax.nn.sigmoid(msg @ a_W + a_b) * msg
    agg_node = jax.ops.segment_sum(msg, dst, num_segments=N)
    x_pool = _silu(jnp.concatenate([x, agg_node], axis=-1) @ u_W1 + u_b1) @ u_W2 + u_b2 + x

    i = src; j = dst
    t2 = jnp.concatenate([x_pool, x], axis=1)
    g2d = _sc_gather_rows(t2, dst, E)
    x_pool_j = g2d[:, :F]
    x_j = g2d[:, F:]
    X_q = jax.ops.segment_max(x_pool_j, i, num_segments=N)
    X_q = jnp.where(jnp.isfinite(X_q), X_q, 0.0)
    mq_t = X_q @ Wq + bq
    M_q = _sc_gather_rows(mq_t, src, E)
    score = (jnp.concatenate([M_q, x_pool_j], axis=-1) @ Watt + batt)[:, 0]
    score = jax.nn.leaky_relu(score, NEG_SLOPE)
    smax = jax.ops.segment_max(score, i, num_segments=N)
    smax = jnp.where(jnp.isfinite(smax), smax, 0.0)
    ex = jnp.exp(score - smax[i])
    denom = jax.ops.segment_sum(ex, i, num_segments=N)
    attn = ex / (denom[i] + 1e-16)
    x_agg = jax.ops.segment_sum(x_j * attn[:, None], i, num_segments=N)

    ew = jnp.where(i == j, 0.0, 1.0).astype(x.dtype)
    deg = jax.ops.segment_sum(ew, i, num_segments=N)
    aggr = jax.ops.segment_sum(ew[:, None] * (x_agg @ le_w)[j], i, num_segments=N)
    le_out = deg[:, None] * (x_agg @ le_W1 + le_b1) + aggr + (x_agg @ le_W2 + le_b2)
    fitness = jax.nn.sigmoid(le_out)[:, 0]

    k = int(np.ceil(RATIO * N))
    topv, perm = jax.lax.top_k(fitness, k)
    x_out = pl.pallas_call(
        _scale_body,
        out_shape=jax.ShapeDtypeStruct((k, F), jnp.float32),
    )(x_agg[perm], topv[:, None])
    return x_out, fitness, perm


# speedup vs baseline: 2.2263x; 1.6898x over previous
import functools

import jax
import jax.numpy as jnp
import numpy as np
from jax import lax
from jax.experimental import pallas as pl
from jax.experimental.pallas import tpu as pltpu
from jax.experimental.pallas import tpu_sc as plsc

N = 10000
E = 320000
F = 128
RATIO = 0.3
NEG_SLOPE = 0.2
EPS = 1e-8

_NC = 2
_NS = 16
_NW = _NC * _NS
_C = 128


def _layer_norm(v, g, b, eps=1e-5):
    mu = jnp.mean(v, axis=-1, keepdims=True)
    var = jnp.var(v, axis=-1, keepdims=True)
    return (v - mu) / jnp.sqrt(var + eps) * g + b


def _silu(v):
    return v * jax.nn.sigmoid(v)


@functools.partial(jax.jit, static_argnums=(2,))
def _sc_gather_rows(table, idx_list, n_out):
    rows, D = table.shape
    n_idx = len(idx_list)
    nblocks = n_out // _C
    iters = (nblocks + _NW - 1) // _NW
    mesh = plsc.VectorSubcoreMesh(core_axis_name="c", subcore_axis_name="s")

    def body(*refs):
        table_hbm = refs[0]
        idx_hbms = refs[1:1 + n_idx]
        out_hbms = refs[1 + n_idx:1 + 2 * n_idx]
        idx_v, rows_v, sem = refs[1 + 2 * n_idx:]
        wid = lax.axis_index("s") * _NC + lax.axis_index("c")

        def step(g, carry):
            blk = g * _NW + wid

            @pl.when(blk < nblocks)
            def _():
                off = blk * _C
                for q in range(n_idx):
                    pltpu.sync_copy(idx_hbms[q].at[pl.ds(off, _C)], idx_v)
                    pltpu.async_copy(table_hbm.at[idx_v], rows_v, sem).wait()
                    pltpu.sync_copy(rows_v, out_hbms[q].at[pl.ds(off, _C)])

            return carry

        lax.fori_loop(0, iters, step, 0)

    out = pl.kernel(
        body,
        out_type=tuple(jax.ShapeDtypeStruct((n_out, D), jnp.float32) for _ in range(n_idx)),
        mesh=mesh,
        scratch_types=[
            pltpu.VMEM((_C,), jnp.int32),
            pltpu.VMEM((_C, D), jnp.float32),
            pltpu.SemaphoreType.DMA,
        ],
    )(table, *idx_list)
    return out


_CS = 2000


@functools.partial(jax.jit, static_argnums=(2,))
def _sc_gather_scalars(tables, idx_list, n_out):
    n_idx = len(idx_list)
    n_tbl = tables[0].shape[0]
    per_w = n_out // _NW
    nchunks = per_w // _CS
    mesh = plsc.VectorSubcoreMesh(core_axis_name="c", subcore_axis_name="s")

    assert n_tbl % 16 == 0

    def body(*refs):
        tbl_hbms = refs[0:n_idx]
        idx_hbms = refs[n_idx:2 * n_idx]
        out_hbms = refs[2 * n_idx:3 * n_idx]
        tbl_vs = refs[3 * n_idx:4 * n_idx]
        idx_v = refs[4 * n_idx]
        out_v = refs[4 * n_idx + 1]
        wid = lax.axis_index("s") * _NC + lax.axis_index("c")
        for q in range(n_idx):
            pltpu.sync_copy(tbl_hbms[q], tbl_vs[q])
        base = wid * per_w

        def chunk(c, carry):
            off = base + c * _CS
            for q in range(n_idx):
                pltpu.sync_copy(idx_hbms[q].at[pl.ds(off, _CS)], idx_v)

                def vec(t, carry2):
                    ids = idx_v[pl.ds(t * 16, 16)]
                    hi = lax.shift_right_logical(ids, 4)
                    lo = lax.bitwise_and(ids, jnp.int32(15))
                    out_v[pl.ds(t * 16, 16)] = plsc.load_gather(tbl_vs[q], [hi, lo])
                    return carry2

                lax.fori_loop(0, _CS // 16, vec, 0)
                pltpu.sync_copy(out_v, out_hbms[q].at[pl.ds(off, _CS)])
            return carry

        lax.fori_loop(0, nchunks, chunk, 0)

    out = pl.kernel(
        body,
        out_type=tuple(jax.ShapeDtypeStruct((n_out,), jnp.float32) for _ in range(n_idx)),
        mesh=mesh,
        scratch_types=[*(pltpu.VMEM((n_tbl // 16, 16), jnp.float32) for _ in range(n_idx)),
                       pltpu.VMEM((_CS,), jnp.int32),
                       pltpu.VMEM((_CS,), jnp.float32)],
        compiler_params=pltpu.CompilerParams(needs_layout_passes=False),
    )(*(t.reshape(n_tbl // 16, 16) for t in tables), *idx_list)
    return out


def _scale_body(xg_ref, tv_ref, o_ref):
    o_ref[...] = xg_ref[...] * tv_ref[...]


def kernel(x, pos, edge_index, Wq, bq, Watt, batt, le_w, le_W1, le_b1, le_W2, le_b2, m_W1, m_b1, m_W2, m_b2, a_W, a_b, u_W1, u_b1, u_W2, u_b2, p_W1, p_b1, p_W2, ln_g, ln_b, coors_scale):
    src = edge_index[0]; dst = edge_index[1]

    ln_x = _layer_norm(x, ln_g, ln_b)
    t1 = jnp.concatenate([ln_x, pos, jnp.zeros((N, 125), jnp.float32)], axis=1)
    g1d, g1s = _sc_gather_rows(t1, (dst, src), E)
    ln_i = g1d[:, :F]; ln_j = g1s[:, :F]
    pos_dir = g1d[:, F:F + 3] - g1s[:, F:F + 3]
    dist = jnp.linalg.norm(pos_dir, axis=-1, keepdims=True)
    m_in = jnp.concatenate([ln_i, ln_j, dist], axis=-1)
    h = _silu(m_in @ m_W1 + m_b1)
    msg = _silu(h @ m_W2 + m_b2)
    msg = jax.nn.sigmoid(msg @ a_W + a_b) * msg
    agg_node = jax.ops.segment_sum(msg, dst, num_segments=N)
    x_pool = _silu(jnp.concatenate([x, agg_node], axis=-1) @ u_W1 + u_b1) @ u_W2 + u_b2 + x

    i = src; j = dst
    t2 = jnp.concatenate([x_pool, x], axis=1)
    (g2d,) = _sc_gather_rows(t2, (dst,), E)
    x_pool_j = g2d[:, :F]
    x_j = g2d[:, F:]
    X_q = jax.ops.segment_max(x_pool_j, i, num_segments=N)
    X_q = jnp.where(jnp.isfinite(X_q), X_q, 0.0)
    mq_t = X_q @ Wq + bq
    (M_q,) = _sc_gather_rows(mq_t, (src,), E)
    score = (jnp.concatenate([M_q, x_pool_j], axis=-1) @ Watt + batt)[:, 0]
    score = jax.nn.leaky_relu(score, NEG_SLOPE)
    smax = jax.ops.segment_max(score, i, num_segments=N)
    smax = jnp.where(jnp.isfinite(smax), smax, 0.0)
    (smax_e,) = _sc_gather_scalars((smax,), (src,), E)
    ex = jnp.exp(score - smax_e)
    denom = jax.ops.segment_sum(ex, i, num_segments=N)
    (denom_e,) = _sc_gather_scalars((denom,), (src,), E)
    attn = ex / (denom_e + 1e-16)
    x_agg = jax.ops.segment_sum(x_j * attn[:, None], i, num_segments=N)

    ew = jnp.where(i == j, 0.0, 1.0).astype(x.dtype)
    deg = jax.ops.segment_sum(ew, i, num_segments=N)
    lw = (x_agg @ le_w)[:, 0]
    (lw_e,) = _sc_gather_scalars((lw,), (dst,), E)
    aggr = jax.ops.segment_sum(ew[:, None] * lw_e[:, None], i, num_segments=N)
    le_out = deg[:, None] * (x_agg @ le_W1 + le_b1) + aggr + (x_agg @ le_W2 + le_b2)
    fitness = jax.nn.sigmoid(le_out)[:, 0]

    k = int(np.ceil(RATIO * N))
    topv, perm = jax.lax.top_k(fitness, k)
    x_out = pl.pallas_call(
        _scale_body,
        out_shape=jax.ShapeDtypeStruct((k, F), jnp.float32),
    )(x_agg[perm], topv[:, None])
    return x_out, fitness, perm
